# Initial kernel scaffold; baseline (speedup 1.0000x reference)
import jax
import jax.numpy as jnp
from jax.experimental import pallas as pl


def kernel(x, edge_index, batch, W_lin1, b_lin1, W_conv0, b_conv0, W_conv1, b_conv1, W_conv2, b_conv2, W_mlp1, b_mlp1, W_mlp2, b_mlp2):
    raise NotImplementedError("write your pallas kernel here")


# speedup vs baseline: 16.2255x; 16.2255x over previous
import functools

import jax
import jax.numpy as jnp
from jax import lax
from jax.experimental import pallas as pl
from jax.experimental.pallas import tpu as pltpu
from jax.experimental.pallas import tpu_sc as plsc

N = 10000
E = 320000
H = 128
G = 64
C = 10
NH = float(N * H)
EPS = 1e-5

NC = 2
NS = 16
NW = NC * NS
K = 100
EPT = E // NW
NCH = EPT // K
SLAB = N // NS

NB = 10
BLK = N // NB

SLAB_A = 624
SLAB_B = N - 15 * SLAB_A

_f32 = jnp.float32


def _zero_slab(zeros_hbm, acc, sid):
    @pl.when(sid < NS - 1)
    def _():
        pltpu.sync_copy(zeros_hbm.at[pl.ds(0, SLAB_A)],
                        acc.at[pl.ds(sid * SLAB_A, SLAB_A)])

    @pl.when(sid == NS - 1)
    def _():
        pltpu.sync_copy(zeros_hbm, acc.at[pl.ds(sid * SLAB_A, SLAB_B)])


def _drain_slab(acc, out_hbm, cid, sid):
    @pl.when(sid < NS - 1)
    def _():
        pltpu.sync_copy(acc.at[pl.ds(sid * SLAB_A, SLAB_A)],
                        out_hbm.at[pl.ds(cid * N + sid * SLAB_A, SLAB_A)])

    @pl.when(sid == NS - 1)
    def _():
        pltpu.sync_copy(acc.at[pl.ds(sid * SLAB_A, SLAB_B)],
                        out_hbm.at[pl.ds(cid * N + sid * SLAB_A, SLAB_B)])


def _sc_deg_body(col_hbm, ones_hbm, zeros_hbm, out_hbm, colv, onesv, acc):
    cid = lax.axis_index("c")
    sid = lax.axis_index("s")
    wid = cid * NS + sid
    pltpu.sync_copy(col_hbm.at[wid], colv)
    pltpu.sync_copy(ones_hbm, onesv)
    _zero_slab(zeros_hbm, acc, sid)
    plsc.subcore_barrier()

    def step(j, carry):
        pltpu.sync_copy(onesv, acc.at[colv.at[j]], add=True)
        return carry

    lax.fori_loop(0, NCH, step, 0)
    plsc.subcore_barrier()
    _drain_slab(acc, out_hbm, cid, sid)


def _sc_scatter_body(u_hbm, row_hbm, col_hbm, zeros_hbm, out_hbm, rowv, colv, buf, acc):
    cid = lax.axis_index("c")
    sid = lax.axis_index("s")
    wid = cid * NS + sid
    pltpu.sync_copy(row_hbm.at[wid], rowv)
    pltpu.sync_copy(col_hbm.at[wid], colv)
    _zero_slab(zeros_hbm, acc, sid)
    plsc.subcore_barrier()

    def step(j, carry):
        pltpu.sync_copy(u_hbm.at[rowv.at[j]], buf)
        pltpu.sync_copy(buf, acc.at[colv.at[j]], add=True)
        return carry

    lax.fori_loop(0, NCH, step, 0)
    plsc.subcore_barrier()
    _drain_slab(acc, out_hbm, cid, sid)


@functools.lru_cache(maxsize=1)
def _build_sc():
    mesh = plsc.VectorSubcoreMesh(
        core_axis_name="c", subcore_axis_name="s",
        num_cores=NC, num_subcores=NS)
    deg = pl.kernel(
        _sc_deg_body,
        out_type=jax.ShapeDtypeStruct((NC * N, H), _f32),
        mesh=mesh,
        scratch_types=[
            pltpu.VMEM((NCH, K), jnp.int32),
            pltpu.VMEM((K, H), _f32),
            pltpu.VMEM_SHARED((N, H), _f32),
        ],
    )
    scat = pl.kernel(
        _sc_scatter_body,
        out_type=jax.ShapeDtypeStruct((NC * N, H), _f32),
        mesh=mesh,
        scratch_types=[
            pltpu.VMEM((NCH, K), jnp.int32),
            pltpu.VMEM((NCH, K), jnp.int32),
            pltpu.VMEM((K, H), _f32),
            pltpu.VMEM_SHARED((N, H), _f32),
        ],
    )
    return deg, scat


def _sc_deg(col2, ones16, zeros16):
    return _build_sc()[0](col2, ones16, zeros16)


def _sc_scatter(u, row2, col2, zerosH):
    return _build_sc()[1](u, row2, col2, zerosH)


def _first_body(x_ref, d_ref, w1_ref, b1_ref, w0_ref, u_ref, dv_ref):
    d = d_ref[...]
    deg = 1.0 + d[0, :, 0:1] + d[1, :, 0:1]
    dv = jnp.where(deg > 0, lax.rsqrt(deg), 0.0)
    h = jnp.maximum(
        jnp.dot(x_ref[...], w1_ref[...], preferred_element_type=_f32)
        + b1_ref[...][None, :], 0.0)
    u_ref[...] = dv * jnp.dot(h, w0_ref[...], preferred_element_type=_f32)
    dv_ref[...] = dv


def _tc_first(x, deg3, W1, b1, W0):
    return pl.pallas_call(
        _first_body,
        grid=(NB,),
        in_specs=[
            pl.BlockSpec((BLK, H), lambda b: (b, 0)),
            pl.BlockSpec((2, BLK, H), lambda b: (0, b, 0)),
            pl.BlockSpec((H, H), lambda b: (0, 0)),
            pl.BlockSpec((H,), lambda b: (0,)),
            pl.BlockSpec((H, H), lambda b: (0, 0)),
        ],
        out_specs=[
            pl.BlockSpec((BLK, H), lambda b: (b, 0)),
            pl.BlockSpec((BLK, 1), lambda b: (b, 0)),
        ],
        out_shape=[
            jax.ShapeDtypeStruct((N, H), _f32),
            jax.ShapeDtypeStruct((N, 1), _f32),
        ],
    )(x, deg3, W1, b1, W0)


def _combine_body(s_ref, u_ref, dv_ref, b_ref, z_ref, st_ref):
    s = s_ref[0] + s_ref[1] + u_ref[...]
    z = jnp.maximum(dv_ref[...] * s + b_ref[...][None, :], 0.0)
    z_ref[...] = z
    st_ref[...] = jnp.stack([jnp.sum(z), jnp.sum(z * z)]).reshape(1, 1, 2)


def _tc_combine(S3, u, dv, bias):
    return pl.pallas_call(
        _combine_body,
        grid=(NB,),
        in_specs=[
            pl.BlockSpec((2, BLK, H), lambda b: (0, b, 0)),
            pl.BlockSpec((BLK, H), lambda b: (b, 0)),
            pl.BlockSpec((BLK, 1), lambda b: (b, 0)),
            pl.BlockSpec((H,), lambda b: (0,)),
        ],
        out_specs=[
            pl.BlockSpec((BLK, H), lambda b: (b, 0)),
            pl.BlockSpec((1, 1, 2), lambda b: (b, 0, 0)),
        ],
        out_shape=[
            jax.ShapeDtypeStruct((N, H), _f32),
            jax.ShapeDtypeStruct((NB, 1, 2), _f32),
        ],
    )(S3, u, dv, bias)


def _next_body(z_ref, st_ref, dv_ref, w_ref, u_ref):
    tot = jnp.sum(st_ref[...], axis=(0, 1))
    m = tot[0] / NH
    v = tot[1] / NH - m * m
    inv = lax.rsqrt(v + EPS)
    hn = (z_ref[...] - m) * inv
    u_ref[...] = dv_ref[...] * jnp.dot(hn, w_ref[...], preferred_element_type=_f32)


def _tc_next(z, st3, dv, W):
    return pl.pallas_call(
        _next_body,
        grid=(NB,),
        in_specs=[
            pl.BlockSpec((BLK, H), lambda b: (b, 0)),
            pl.BlockSpec((NB, 1, 2), lambda b: (0, 0, 0)),
            pl.BlockSpec((BLK, 1), lambda b: (b, 0)),
            pl.BlockSpec((H, H), lambda b: (0, 0)),
        ],
        out_specs=pl.BlockSpec((BLK, H), lambda b: (b, 0)),
        out_shape=jax.ShapeDtypeStruct((N, H), _f32),
    )(z, st3, dv, W)


def _head_body(z_ref, st_ref, bt_ref, w1_ref, b1_ref, w2_ref, b2_ref, o_ref):
    tot = jnp.sum(st_ref[...], axis=(0, 1))
    m = tot[0] / NH
    v = tot[1] / NH - m * m
    inv = lax.rsqrt(v + EPS)

    def step(c, carry):
        pool, cnt = carry
        zc = z_ref[pl.ds(c * BLK, BLK), :]
        hn = (zc - m) * inv
        bc = bt_ref[c, :]
        gi = lax.broadcasted_iota(jnp.int32, (G, BLK), 0)
        oh = (bc[None, :] == gi).astype(_f32)
        pool = pool + jnp.dot(oh, hn, preferred_element_type=_f32)
        cnt = cnt + jnp.sum(oh, axis=1, keepdims=True)
        return pool, cnt

    pool, cnt = lax.fori_loop(
        0, NB, step,
        (jnp.zeros((G, H), _f32), jnp.zeros((G, 1), _f32)))
    pooled = pool / jnp.maximum(cnt, 1.0)
    a = jnp.maximum(
        jnp.dot(pooled, w1_ref[...], preferred_element_type=_f32)
        + b1_ref[...][None, :], 0.0)
    sc = jnp.dot(a, w2_ref[...], preferred_element_type=_f32) + b2_ref[...][None, :]
    mx = jnp.max(sc, axis=-1, keepdims=True)
    o_ref[...] = sc - (jnp.log(jnp.sum(jnp.exp(sc - mx), axis=-1, keepdims=True)) + mx)


def _tc_head(z, st3, batch2, W1, b1, W2, b2):
    return pl.pallas_call(
        _head_body,
        out_shape=jax.ShapeDtypeStruct((G, C), _f32),
    )(z, st3, batch2, W1, b1, W2, b2)


def kernel(x, edge_index, batch, W_lin1, b_lin1, W_conv0, b_conv0, W_conv1,
           b_conv1, W_conv2, b_conv2, W_mlp1, b_mlp1, W_mlp2, b_mlp2):
    ei = edge_index.astype(jnp.int32)
    row2 = ei[0].reshape(NW, NCH, K)
    col2 = ei[1].reshape(NW, NCH, K)
    batch2 = batch.astype(jnp.int32).reshape(NB, BLK)

    onesH = jnp.ones((K, H), _f32)
    zerosH = jnp.zeros((SLAB_B, H), _f32)

    deg3 = _sc_deg(col2, onesH, zerosH).reshape(2, N, H)
    u, dv = _tc_first(x, deg3, W_lin1, b_lin1, W_conv0)

    biases = (b_conv0, b_conv1, b_conv2)
    nextW = (W_conv1, W_conv2)
    z = st3 = None
    for i in range(3):
        S3 = _sc_scatter(u, row2, col2, zerosH).reshape(2, N, H)
        z, st3 = _tc_combine(S3, u, dv, biases[i])
        if i < 2:
            u = _tc_next(z, st3, dv, nextW[i])

    return _tc_head(z, st3, batch2, W_mlp1, b_mlp1, W_mlp2, b_mlp2)


# speedup vs baseline: 22.8734x; 1.4097x over previous
import functools

import jax
import jax.numpy as jnp
from jax import lax
from jax.experimental import pallas as pl
from jax.experimental.pallas import tpu as pltpu
from jax.experimental.pallas import tpu_sc as plsc

N = 10000
E = 320000
H = 128
G = 64
C = 10
NH = float(N * H)
EPS = 1e-5

NC = 2
NS = 16
NW = NC * NS
K = 100
EPT = E // NW
NCH = EPT // K
SLAB = N // NS
NPH = 2
NCH2 = NCH // NPH

NB = 10
BLK = N // NB

SLAB_A = 624
SLAB_B = N - 15 * SLAB_A

_f32 = jnp.float32


def _zero_slab(zeros_hbm, acc, sid):
    @pl.when(sid < NS - 1)
    def _():
        pltpu.sync_copy(zeros_hbm.at[pl.ds(0, SLAB_A)],
                        acc.at[pl.ds(sid * SLAB_A, SLAB_A)])

    @pl.when(sid == NS - 1)
    def _():
        pltpu.sync_copy(zeros_hbm, acc.at[pl.ds(sid * SLAB_A, SLAB_B)])


def _drain_slab(acc, out_hbm, cid, sid):
    @pl.when(sid < NS - 1)
    def _():
        pltpu.sync_copy(acc.at[pl.ds(sid * SLAB_A, SLAB_A)],
                        out_hbm.at[pl.ds(cid * N + sid * SLAB_A, SLAB_A)])

    @pl.when(sid == NS - 1)
    def _():
        pltpu.sync_copy(acc.at[pl.ds(sid * SLAB_A, SLAB_B)],
                        out_hbm.at[pl.ds(cid * N + sid * SLAB_A, SLAB_B)])


def _sc_deg_body(col_hbm, ones_hbm, zeros_hbm, out_hbm, colv, onesv, sem, acc):
    cid = lax.axis_index("c")
    sid = lax.axis_index("s")
    wid = cid * NS + sid
    pltpu.sync_copy(col_hbm.at[wid], colv)
    pltpu.sync_copy(ones_hbm, onesv)
    _zero_slab(zeros_hbm, acc, sid)
    plsc.subcore_barrier()

    def step(t, carry):
        for q in range(4):
            pltpu.async_copy(onesv, acc.at[colv.at[4 * t + q]], sem, add=True)
        for q in range(4):
            pltpu.make_async_copy(onesv, acc.at[colv.at[4 * t + q]], sem).wait()
        return carry

    lax.fori_loop(0, NCH // 4, step, 0)
    plsc.subcore_barrier()
    _drain_slab(acc, out_hbm, cid, sid)


def _sc_scatter_body(u_hbm, row_hbm, col_hbm, zeros_hbm, out_hbm, rowv, colv,
                     buf0, buf1, sem0, sem1, acc):
    cid = lax.axis_index("c")
    sid = lax.axis_index("s")
    wid = cid * NS + sid
    pltpu.sync_copy(row_hbm.at[wid], rowv)
    pltpu.sync_copy(col_hbm.at[wid], colv)
    _zero_slab(zeros_hbm, acc, sid)
    plsc.subcore_barrier()

    for p in range(NPH):
        pltpu.sync_copy(row_hbm.at[wid * NPH + p], rowv)
        pltpu.sync_copy(col_hbm.at[wid * NPH + p], colv)
        pltpu.async_copy(u_hbm.at[rowv.at[0]], buf0, sem0)

        def step(t, carry):
            j0 = 2 * t
            j1 = 2 * t + 1
            j2 = jnp.minimum(2 * t + 2, NCH2 - 1)
            pltpu.async_copy(u_hbm.at[rowv.at[j1]], buf1, sem1)
            pltpu.make_async_copy(u_hbm.at[rowv.at[j0]], buf0, sem0).wait()
            pltpu.sync_copy(buf0, acc.at[colv.at[j0]], add=True)
            pltpu.async_copy(u_hbm.at[rowv.at[j2]], buf0, sem0)
            pltpu.make_async_copy(u_hbm.at[rowv.at[j1]], buf1, sem1).wait()
            pltpu.sync_copy(buf1, acc.at[colv.at[j1]], add=True)
            return carry

        lax.fori_loop(0, NCH2 // 2, step, 0)
        pltpu.make_async_copy(u_hbm.at[rowv.at[NCH2 - 1]], buf0, sem0).wait()
    plsc.subcore_barrier()
    _drain_slab(acc, out_hbm, cid, sid)


@functools.lru_cache(maxsize=1)
def _build_sc():
    mesh = plsc.VectorSubcoreMesh(
        core_axis_name="c", subcore_axis_name="s",
        num_cores=NC, num_subcores=NS)
    deg = pl.kernel(
        _sc_deg_body,
        out_type=jax.ShapeDtypeStruct((NC * N, H), _f32),
        mesh=mesh,
        scratch_types=[
            pltpu.VMEM((NCH, K), jnp.int32),
            pltpu.VMEM((K, H), _f32),
            pltpu.SemaphoreType.DMA,
            pltpu.VMEM_SHARED((N, H), _f32),
        ],
    )
    scat = pl.kernel(
        _sc_scatter_body,
        out_type=jax.ShapeDtypeStruct((NC * N, H), _f32),
        mesh=mesh,
        scratch_types=[
            pltpu.VMEM((NCH2, K), jnp.int32),
            pltpu.VMEM((NCH2, K), jnp.int32),
            pltpu.VMEM((K, H), _f32),
            pltpu.VMEM((K, H), _f32),
            pltpu.SemaphoreType.DMA,
            pltpu.SemaphoreType.DMA,
            pltpu.VMEM_SHARED((N, H), _f32),
        ],
    )
    return deg, scat


def _sc_deg(col2, ones16, zeros16):
    return _build_sc()[0](col2, ones16, zeros16)


def _sc_scatter(u, row2, col2, zerosH):
    return _build_sc()[1](u, row2, col2, zerosH)


def _first_body(x_ref, d_ref, w1_ref, b1_ref, w0_ref, u_ref, dv_ref):
    d = d_ref[...]
    deg = 1.0 + d[0, :, 0:1] + d[1, :, 0:1]
    dv = jnp.where(deg > 0, lax.rsqrt(deg), 0.0)
    h = jnp.maximum(
        jnp.dot(x_ref[...], w1_ref[...], preferred_element_type=_f32)
        + b1_ref[...][None, :], 0.0)
    u_ref[...] = dv * jnp.dot(h, w0_ref[...], preferred_element_type=_f32)
    dv_ref[...] = dv


def _tc_first(x, deg3, W1, b1, W0):
    return pl.pallas_call(
        _first_body,
        grid=(NB,),
        in_specs=[
            pl.BlockSpec((BLK, H), lambda b: (b, 0)),
            pl.BlockSpec((2, BLK, H), lambda b: (0, b, 0)),
            pl.BlockSpec((H, H), lambda b: (0, 0)),
            pl.BlockSpec((H,), lambda b: (0,)),
            pl.BlockSpec((H, H), lambda b: (0, 0)),
        ],
        out_specs=[
            pl.BlockSpec((BLK, H), lambda b: (b, 0)),
            pl.BlockSpec((BLK, 1), lambda b: (b, 0)),
        ],
        out_shape=[
            jax.ShapeDtypeStruct((N, H), _f32),
            jax.ShapeDtypeStruct((N, 1), _f32),
        ],
    )(x, deg3, W1, b1, W0)


def _combine_body(s_ref, u_ref, dv_ref, b_ref, z_ref, st_ref):
    s = s_ref[0] + s_ref[1] + u_ref[...]
    z = jnp.maximum(dv_ref[...] * s + b_ref[...][None, :], 0.0)
    z_ref[...] = z
    st_ref[...] = jnp.stack([jnp.sum(z), jnp.sum(z * z)]).reshape(1, 1, 2)


def _tc_combine(S3, u, dv, bias):
    return pl.pallas_call(
        _combine_body,
        grid=(NB,),
        in_specs=[
            pl.BlockSpec((2, BLK, H), lambda b: (0, b, 0)),
            pl.BlockSpec((BLK, H), lambda b: (b, 0)),
            pl.BlockSpec((BLK, 1), lambda b: (b, 0)),
            pl.BlockSpec((H,), lambda b: (0,)),
        ],
        out_specs=[
            pl.BlockSpec((BLK, H), lambda b: (b, 0)),
            pl.BlockSpec((1, 1, 2), lambda b: (b, 0, 0)),
        ],
        out_shape=[
            jax.ShapeDtypeStruct((N, H), _f32),
            jax.ShapeDtypeStruct((NB, 1, 2), _f32),
        ],
    )(S3, u, dv, bias)


def _next_body(z_ref, st_ref, dv_ref, w_ref, u_ref):
    tot = jnp.sum(st_ref[...], axis=(0, 1))
    m = tot[0] / NH
    v = tot[1] / NH - m * m
    inv = lax.rsqrt(v + EPS)
    hn = (z_ref[...] - m) * inv
    u_ref[...] = dv_ref[...] * jnp.dot(hn, w_ref[...], preferred_element_type=_f32)


def _tc_next(z, st3, dv, W):
    return pl.pallas_call(
        _next_body,
        grid=(NB,),
        in_specs=[
            pl.BlockSpec((BLK, H), lambda b: (b, 0)),
            pl.BlockSpec((NB, 1, 2), lambda b: (0, 0, 0)),
            pl.BlockSpec((BLK, 1), lambda b: (b, 0)),
            pl.BlockSpec((H, H), lambda b: (0, 0)),
        ],
        out_specs=pl.BlockSpec((BLK, H), lambda b: (b, 0)),
        out_shape=jax.ShapeDtypeStruct((N, H), _f32),
    )(z, st3, dv, W)


def _head_body(z_ref, st_ref, bt_ref, w1_ref, b1_ref, w2_ref, b2_ref, o_ref):
    tot = jnp.sum(st_ref[...], axis=(0, 1))
    m = tot[0] / NH
    v = tot[1] / NH - m * m
    inv = lax.rsqrt(v + EPS)

    def step(c, carry):
        pool, cnt = carry
        zc = z_ref[pl.ds(c * BLK, BLK), :]
        hn = (zc - m) * inv
        bc = bt_ref[c, :]
        gi = lax.broadcasted_iota(jnp.int32, (G, BLK), 0)
        oh = (bc[None, :] == gi).astype(_f32)
        pool = pool + jnp.dot(oh, hn, preferred_element_type=_f32)
        cnt = cnt + jnp.sum(oh, axis=1, keepdims=True)
        return pool, cnt

    pool, cnt = lax.fori_loop(
        0, NB, step,
        (jnp.zeros((G, H), _f32), jnp.zeros((G, 1), _f32)))
    pooled = pool / jnp.maximum(cnt, 1.0)
    a = jnp.maximum(
        jnp.dot(pooled, w1_ref[...], preferred_element_type=_f32)
        + b1_ref[...][None, :], 0.0)
    sc = jnp.dot(a, w2_ref[...], preferred_element_type=_f32) + b2_ref[...][None, :]
    mx = jnp.max(sc, axis=-1, keepdims=True)
    o_ref[...] = sc - (jnp.log(jnp.sum(jnp.exp(sc - mx), axis=-1, keepdims=True)) + mx)


def _tc_head(z, st3, batch2, W1, b1, W2, b2):
    return pl.pallas_call(
        _head_body,
        out_shape=jax.ShapeDtypeStruct((G, C), _f32),
    )(z, st3, batch2, W1, b1, W2, b2)


def kernel(x, edge_index, batch, W_lin1, b_lin1, W_conv0, b_conv0, W_conv1,
           b_conv1, W_conv2, b_conv2, W_mlp1, b_mlp1, W_mlp2, b_mlp2):
    ei = edge_index.astype(jnp.int32)
    row4 = ei[0].reshape(NW * NPH, NCH2, K)
    col4 = ei[1].reshape(NW * NPH, NCH2, K)
    col2 = ei[1].reshape(NW, NCH, K)
    batch2 = batch.astype(jnp.int32).reshape(NB, BLK)

    onesH = jnp.ones((K, H), _f32)
    zerosH = jnp.zeros((SLAB_B, H), _f32)

    deg3 = _sc_deg(col2, onesH, zerosH).reshape(2, N, H)
    u, dv = _tc_first(x, deg3, W_lin1, b_lin1, W_conv0)

    biases = (b_conv0, b_conv1, b_conv2)
    nextW = (W_conv1, W_conv2)
    z = st3 = None
    for i in range(3):
        S3 = _sc_scatter(u, row4, col4, zerosH).reshape(2, N, H)
        z, st3 = _tc_combine(S3, u, dv, biases[i])
        if i < 2:
            u = _tc_next(z, st3, dv, nextW[i])

    return _tc_head(z, st3, batch2, W_mlp1, b_mlp1, W_mlp2, b_mlp2)


# speedup vs baseline: 23.3436x; 1.0206x over previous
import functools

import jax
import jax.numpy as jnp
from jax import lax
from jax.experimental import pallas as pl
from jax.experimental.pallas import tpu as pltpu
from jax.experimental.pallas import tpu_sc as plsc

N = 10000
E = 320000
H = 128
G = 64
C = 10
NH = float(N * H)
EPS = 1e-5

NC = 2
NS = 16
NW = NC * NS
K = 125
EPT = E // NW
NCH = EPT // K
SLAB = N // NS
NPH = 2
NCH2 = NCH // NPH

NB = 10
BLK = N // NB

SLAB_A = 624
SLAB_B = N - 15 * SLAB_A

_f32 = jnp.float32


def _zero_slab(zeros_hbm, acc, sid):
    @pl.when(sid < NS - 1)
    def _():
        pltpu.sync_copy(zeros_hbm.at[pl.ds(0, SLAB_A)],
                        acc.at[pl.ds(sid * SLAB_A, SLAB_A)])

    @pl.when(sid == NS - 1)
    def _():
        pltpu.sync_copy(zeros_hbm, acc.at[pl.ds(sid * SLAB_A, SLAB_B)])


def _drain_slab(acc, out_hbm, cid, sid):
    @pl.when(sid < NS - 1)
    def _():
        pltpu.sync_copy(acc.at[pl.ds(sid * SLAB_A, SLAB_A)],
                        out_hbm.at[pl.ds(cid * N + sid * SLAB_A, SLAB_A)])

    @pl.when(sid == NS - 1)
    def _():
        pltpu.sync_copy(acc.at[pl.ds(sid * SLAB_A, SLAB_B)],
                        out_hbm.at[pl.ds(cid * N + sid * SLAB_A, SLAB_B)])


def _sc_deg_body(col_hbm, ones_hbm, zeros_hbm, out_hbm, colv, onesv, sem, acc):
    cid = lax.axis_index("c")
    sid = lax.axis_index("s")
    wid = cid * NS + sid
    pltpu.sync_copy(col_hbm.at[wid], colv)
    pltpu.sync_copy(ones_hbm, onesv)
    _zero_slab(zeros_hbm, acc, sid)
    plsc.subcore_barrier()

    def step(t, carry):
        for q in range(4):
            pltpu.async_copy(onesv, acc.at[colv.at[4 * t + q]], sem, add=True)
        for q in range(4):
            pltpu.make_async_copy(onesv, acc.at[colv.at[4 * t + q]], sem).wait()
        return carry

    lax.fori_loop(0, NCH // 4, step, 0)
    plsc.subcore_barrier()
    _drain_slab(acc, out_hbm, cid, sid)


def _sc_scatter_body(u_hbm, row_hbm, col_hbm, zeros_hbm, out_hbm, rowv, colv,
                     buf0, buf1, sem0, sem1, acc):
    cid = lax.axis_index("c")
    sid = lax.axis_index("s")
    wid = cid * NS + sid
    pltpu.sync_copy(row_hbm.at[wid], rowv)
    pltpu.sync_copy(col_hbm.at[wid], colv)
    _zero_slab(zeros_hbm, acc, sid)
    plsc.subcore_barrier()

    for p in range(NPH):
        pltpu.sync_copy(row_hbm.at[wid * NPH + p], rowv)
        pltpu.sync_copy(col_hbm.at[wid * NPH + p], colv)
        pltpu.async_copy(u_hbm.at[rowv.at[0]], buf0, sem0)

        def step(t, carry):
            j0 = 2 * t
            j1 = 2 * t + 1
            j2 = jnp.minimum(2 * t + 2, NCH2 - 1)
            pltpu.async_copy(u_hbm.at[rowv.at[j1]], buf1, sem1)
            pltpu.make_async_copy(u_hbm.at[rowv.at[j0]], buf0, sem0).wait()
            pltpu.sync_copy(buf0, acc.at[colv.at[j0]], add=True)
            pltpu.async_copy(u_hbm.at[rowv.at[j2]], buf0, sem0)
            pltpu.make_async_copy(u_hbm.at[rowv.at[j1]], buf1, sem1).wait()
            pltpu.sync_copy(buf1, acc.at[colv.at[j1]], add=True)
            return carry

        lax.fori_loop(0, NCH2 // 2, step, 0)
        pltpu.make_async_copy(u_hbm.at[rowv.at[NCH2 - 1]], buf0, sem0).wait()
    plsc.subcore_barrier()
    _drain_slab(acc, out_hbm, cid, sid)


@functools.lru_cache(maxsize=1)
def _build_sc():
    mesh = plsc.VectorSubcoreMesh(
        core_axis_name="c", subcore_axis_name="s",
        num_cores=NC, num_subcores=NS)
    deg = pl.kernel(
        _sc_deg_body,
        out_type=jax.ShapeDtypeStruct((NC * N, H), _f32),
        mesh=mesh,
        scratch_types=[
            pltpu.VMEM((NCH, K), jnp.int32),
            pltpu.VMEM((K, H), _f32),
            pltpu.SemaphoreType.DMA,
            pltpu.VMEM_SHARED((N, H), _f32),
        ],
    )
    scat = pl.kernel(
        _sc_scatter_body,
        out_type=jax.ShapeDtypeStruct((NC * N, H), _f32),
        mesh=mesh,
        scratch_types=[
            pltpu.VMEM((NCH2, K), jnp.int32),
            pltpu.VMEM((NCH2, K), jnp.int32),
            pltpu.VMEM((K, H), _f32),
            pltpu.VMEM((K, H), _f32),
            pltpu.SemaphoreType.DMA,
            pltpu.SemaphoreType.DMA,
            pltpu.VMEM_SHARED((N, H), _f32),
        ],
    )
    return deg, scat


def _sc_deg(col2, ones16, zeros16):
    return _build_sc()[0](col2, ones16, zeros16)


def _sc_scatter(u, row2, col2, zerosH):
    return _build_sc()[1](u, row2, col2, zerosH)


def _first_body(x_ref, d_ref, w1_ref, b1_ref, w0_ref, u_ref, dv_ref):
    d = d_ref[...]
    deg = 1.0 + d[0, :, 0:1] + d[1, :, 0:1]
    dv = jnp.where(deg > 0, lax.rsqrt(deg), 0.0)
    h = jnp.maximum(
        jnp.dot(x_ref[...], w1_ref[...], preferred_element_type=_f32)
        + b1_ref[...][None, :], 0.0)
    u_ref[...] = dv * jnp.dot(h, w0_ref[...], preferred_element_type=_f32)
    dv_ref[...] = dv


def _tc_first(x, deg3, W1, b1, W0):
    return pl.pallas_call(
        _first_body,
        grid=(NB,),
        in_specs=[
            pl.BlockSpec((BLK, H), lambda b: (b, 0)),
            pl.BlockSpec((2, BLK, H), lambda b: (0, b, 0)),
            pl.BlockSpec((H, H), lambda b: (0, 0)),
            pl.BlockSpec((H,), lambda b: (0,)),
            pl.BlockSpec((H, H), lambda b: (0, 0)),
        ],
        out_specs=[
            pl.BlockSpec((BLK, H), lambda b: (b, 0)),
            pl.BlockSpec((BLK, 1), lambda b: (b, 0)),
        ],
        out_shape=[
            jax.ShapeDtypeStruct((N, H), _f32),
            jax.ShapeDtypeStruct((N, 1), _f32),
        ],
    )(x, deg3, W1, b1, W0)


def _combine_body(s_ref, u_ref, dv_ref, b_ref, z_ref, st_ref):
    s = s_ref[0] + s_ref[1] + u_ref[...]
    z = jnp.maximum(dv_ref[...] * s + b_ref[...][None, :], 0.0)
    z_ref[...] = z
    st_ref[...] = jnp.stack([jnp.sum(z), jnp.sum(z * z)]).reshape(1, 1, 2)


def _tc_combine(S3, u, dv, bias):
    return pl.pallas_call(
        _combine_body,
        grid=(NB,),
        in_specs=[
            pl.BlockSpec((2, BLK, H), lambda b: (0, b, 0)),
            pl.BlockSpec((BLK, H), lambda b: (b, 0)),
            pl.BlockSpec((BLK, 1), lambda b: (b, 0)),
            pl.BlockSpec((H,), lambda b: (0,)),
        ],
        out_specs=[
            pl.BlockSpec((BLK, H), lambda b: (b, 0)),
            pl.BlockSpec((1, 1, 2), lambda b: (b, 0, 0)),
        ],
        out_shape=[
            jax.ShapeDtypeStruct((N, H), _f32),
            jax.ShapeDtypeStruct((NB, 1, 2), _f32),
        ],
    )(S3, u, dv, bias)


def _next_body(z_ref, st_ref, dv_ref, w_ref, u_ref):
    tot = jnp.sum(st_ref[...], axis=(0, 1))
    m = tot[0] / NH
    v = tot[1] / NH - m * m
    inv = lax.rsqrt(v + EPS)
    hn = (z_ref[...] - m) * inv
    u_ref[...] = dv_ref[...] * jnp.dot(hn, w_ref[...], preferred_element_type=_f32)


def _tc_next(z, st3, dv, W):
    return pl.pallas_call(
        _next_body,
        grid=(NB,),
        in_specs=[
            pl.BlockSpec((BLK, H), lambda b: (b, 0)),
            pl.BlockSpec((NB, 1, 2), lambda b: (0, 0, 0)),
            pl.BlockSpec((BLK, 1), lambda b: (b, 0)),
            pl.BlockSpec((H, H), lambda b: (0, 0)),
        ],
        out_specs=pl.BlockSpec((BLK, H), lambda b: (b, 0)),
        out_shape=jax.ShapeDtypeStruct((N, H), _f32),
    )(z, st3, dv, W)


def _head_body(z_ref, st_ref, bt_ref, w1_ref, b1_ref, w2_ref, b2_ref, o_ref):
    tot = jnp.sum(st_ref[...], axis=(0, 1))
    m = tot[0] / NH
    v = tot[1] / NH - m * m
    inv = lax.rsqrt(v + EPS)

    def step(c, carry):
        pool, cnt = carry
        zc = z_ref[pl.ds(c * BLK, BLK), :]
        hn = (zc - m) * inv
        bc = bt_ref[c, :]
        gi = lax.broadcasted_iota(jnp.int32, (G, BLK), 0)
        oh = (bc[None, :] == gi).astype(_f32)
        pool = pool + jnp.dot(oh, hn, preferred_element_type=_f32)
        cnt = cnt + jnp.sum(oh, axis=1, keepdims=True)
        return pool, cnt

    pool, cnt = lax.fori_loop(
        0, NB, step,
        (jnp.zeros((G, H), _f32), jnp.zeros((G, 1), _f32)))
    pooled = pool / jnp.maximum(cnt, 1.0)
    a = jnp.maximum(
        jnp.dot(pooled, w1_ref[...], preferred_element_type=_f32)
        + b1_ref[...][None, :], 0.0)
    sc = jnp.dot(a, w2_ref[...], preferred_element_type=_f32) + b2_ref[...][None, :]
    mx = jnp.max(sc, axis=-1, keepdims=True)
    o_ref[...] = sc - (jnp.log(jnp.sum(jnp.exp(sc - mx), axis=-1, keepdims=True)) + mx)


def _tc_head(z, st3, batch2, W1, b1, W2, b2):
    return pl.pallas_call(
        _head_body,
        out_shape=jax.ShapeDtypeStruct((G, C), _f32),
    )(z, st3, batch2, W1, b1, W2, b2)


def kernel(x, edge_index, batch, W_lin1, b_lin1, W_conv0, b_conv0, W_conv1,
           b_conv1, W_conv2, b_conv2, W_mlp1, b_mlp1, W_mlp2, b_mlp2):
    ei = edge_index.astype(jnp.int32)
    row4 = ei[0].reshape(NW * NPH, NCH2, K)
    col4 = ei[1].reshape(NW * NPH, NCH2, K)
    col2 = ei[1].reshape(NW, NCH, K)
    batch2 = batch.astype(jnp.int32).reshape(NB, BLK)

    onesH = jnp.ones((K, H), _f32)
    zerosH = jnp.zeros((SLAB_B, H), _f32)

    deg3 = _sc_deg(col2, onesH, zerosH).reshape(2, N, H)
    u, dv = _tc_first(x, deg3, W_lin1, b_lin1, W_conv0)

    biases = (b_conv0, b_conv1, b_conv2)
    nextW = (W_conv1, W_conv2)
    z = st3 = None
    for i in range(3):
        S3 = _sc_scatter(u, row4, col4, zerosH).reshape(2, N, H)
        z, st3 = _tc_combine(S3, u, dv, biases[i])
        if i < 2:
            u = _tc_next(z, st3, dv, nextW[i])

    return _tc_head(z, st3, batch2, W_mlp1, b_mlp1, W_mlp2, b_mlp2)
